# baseline (speedup 1.0000x reference)
import functools

import jax
import jax.numpy as jnp
from jax import lax
from jax.experimental import pallas as pl
from jax.experimental.pallas import tpu as pltpu
from jax.experimental.pallas import tpu_sc as plsc

D_MODEL = 1024
VOCAB = 32000
BATCH = 4
SEQ = 2048
IMG_LEN = 256

NTOK = BATCH * SEQ
NSPLIT = 2
PART = NTOK // NSPLIT
NC, NS = 2, 16
NW = NC * NS
PER_W = PART // NW
CHUNK = 64
NCHUNK = PER_W // CHUNK

OUT_ROWS = BATCH * (IMG_LEN + SEQ)
BLK = 256
NBLK = 2
BN = D_MODEL // NBLK
BPB = (IMG_LEN + SEQ) // BLK
IMG_BLOCKS = BATCH * IMG_LEN // BLK
EMB_BLOCKS_P = PART // BLK


@functools.lru_cache(maxsize=None)
def _build_gather(part: int):
    mesh = plsc.VectorSubcoreMesh(core_axis_name="c", subcore_axis_name="s")

    @functools.partial(
        pl.kernel,
        mesh=mesh,
        out_type=jax.ShapeDtypeStruct((PART, D_MODEL), jnp.float32),
        scratch_types=[
            pltpu.VMEM((CHUNK,), jnp.int32),
            pltpu.VMEM((CHUNK, D_MODEL), jnp.float32),
            pltpu.SemaphoreType.DMA,
        ],
    )
    def _gather(ids_hbm, table_hbm, out_hbm, idx_v, rows_v, sem):
        wid = lax.axis_index("s") * NC + lax.axis_index("c")
        base = wid * PER_W
        for c in range(NCHUNK):
            off = base + c * CHUNK
            pltpu.sync_copy(ids_hbm.at[pl.ds(part * PART + off, CHUNK)], idx_v)
            pltpu.async_copy(table_hbm.at[idx_v], rows_v, sem).wait()
            pltpu.sync_copy(rows_v, out_hbm.at[pl.ds(off, CHUNK)])

    return _gather


def _mm_img_body(img_ref, w_ref, b_ref, out_ref):
    w_bf = w_ref[...].astype(jnp.bfloat16)
    x = img_ref[...].astype(jnp.bfloat16)
    out_ref[...] = (
        jnp.dot(x, w_bf, preferred_element_type=jnp.float32) + b_ref[...]
    )


def _mm_emb_body(prev_ref, emb_ref, w_ref, b_ref, out_ref):
    del prev_ref
    x = emb_ref[...].astype(jnp.bfloat16)
    out_ref[...] = (
        jnp.dot(x, w_ref[...], preferred_element_type=jnp.float32) + b_ref[...]
    )


@functools.lru_cache(maxsize=None)
def _build_mm_img():
    return pl.pallas_call(
        _mm_img_body,
        grid=(IMG_BLOCKS,),
        in_specs=[
            pl.BlockSpec((BLK, D_MODEL), lambda j: (j, 0)),
            pl.BlockSpec((D_MODEL, D_MODEL), lambda j: (0, 0)),
            pl.BlockSpec((1, D_MODEL), lambda j: (0, 0)),
        ],
        out_specs=pl.BlockSpec((BLK, D_MODEL), lambda j: (j * BPB, 0)),
        out_shape=jax.ShapeDtypeStruct((OUT_ROWS, D_MODEL), jnp.float32),
        compiler_params=pltpu.CompilerParams(
            dimension_semantics=("arbitrary",),
        ),
    )


@functools.lru_cache(maxsize=None)
def _build_mm_emb(part: int):
    def out_map(j, n, part=part):
        return ((part * 2 + j // 8) * BPB + 1 + j % 8, n)

    return pl.pallas_call(
        _mm_emb_body,
        grid=(EMB_BLOCKS_P, NBLK),
        in_specs=[
            pl.BlockSpec(memory_space=pl.ANY),
            pl.BlockSpec((BLK, D_MODEL), lambda j, n: (j, 0)),
            pl.BlockSpec((D_MODEL, BN), lambda j, n: (0, n)),
            pl.BlockSpec((1, BN), lambda j, n: (0, n)),
        ],
        out_specs=pl.BlockSpec((BLK, BN), out_map),
        out_shape=jax.ShapeDtypeStruct((OUT_ROWS, D_MODEL), jnp.float32),
        input_output_aliases={0: 0},
        compiler_params=pltpu.CompilerParams(
            dimension_semantics=("arbitrary", "arbitrary"),
        ),
    )


def kernel(input_ids, image_features, table, W, b):
    ids_flat = input_ids.reshape(NTOK)
    embs = [_build_gather(q)(ids_flat, table) for q in range(NSPLIT)]
    img2d = image_features.reshape(BATCH * IMG_LEN, D_MODEL)
    b2d = b.reshape(1, D_MODEL)
    w_bf = W.astype(jnp.bfloat16)
    out = _build_mm_img()(img2d, W, b2d)
    for q in range(NSPLIT):
        out = _build_mm_emb(q)(out, embs[q], w_bf, b2d)
    return out.reshape(BATCH, IMG_LEN + SEQ, D_MODEL)


# speedup vs baseline: 1.4390x; 1.4390x over previous
import functools

import jax
import jax.numpy as jnp
from jax import lax
from jax.experimental import pallas as pl
from jax.experimental.pallas import tpu as pltpu
from jax.experimental.pallas import tpu_sc as plsc

D_MODEL = 1024
VOCAB = 32000
BATCH = 4
SEQ = 2048
IMG_LEN = 256

NTOK = BATCH * SEQ
NSPLIT = 2
PART = NTOK // NSPLIT
NC, NS = 2, 16
NW = NC * NS
PER_W = PART // NW
CHUNK = 64
NCHUNK = PER_W // CHUNK

OUT_ROWS = BATCH * (IMG_LEN + SEQ)
BLK = 256
NBLK = 2
BN = D_MODEL // NBLK
BPB = (IMG_LEN + SEQ) // BLK
IMG_BLOCKS = BATCH * IMG_LEN // BLK
EMB_BLOCKS_P = PART // BLK


@functools.lru_cache(maxsize=None)
def _build_gather(part: int):
    mesh = plsc.VectorSubcoreMesh(core_axis_name="c", subcore_axis_name="s")

    @functools.partial(
        pl.kernel,
        mesh=mesh,
        out_type=jax.ShapeDtypeStruct((PART, D_MODEL), jnp.float32),
        scratch_types=[
            pltpu.VMEM((CHUNK,), jnp.int32),
            pltpu.VMEM((CHUNK, D_MODEL), jnp.float32),
            pltpu.SemaphoreType.DMA,
        ],
    )
    def _gather(ids_hbm, table_hbm, out_hbm, idx_v, rows_v, sem):
        wid = lax.axis_index("s") * NC + lax.axis_index("c")
        base = wid * PER_W
        for c in range(NCHUNK):
            off = base + c * CHUNK
            pltpu.sync_copy(ids_hbm.at[pl.ds(part * PART + off, CHUNK)], idx_v)
            pltpu.async_copy(table_hbm.at[idx_v], rows_v, sem).wait()
            pltpu.sync_copy(rows_v, out_hbm.at[pl.ds(off, CHUNK)])

    return _gather


def _mm_img_body(img_ref, w_ref, b_ref, out_ref, wbf_ref):
    w_bf = w_ref[...].astype(jnp.bfloat16)
    wbf_ref[...] = w_bf
    x = img_ref[...].astype(jnp.bfloat16)
    out_ref[...] = (
        jnp.dot(x, w_bf, preferred_element_type=jnp.float32) + b_ref[...]
    )


def _mm_emb_body(prev_ref, emb_ref, w_ref, b_ref, out_ref):
    del prev_ref
    x = emb_ref[...].astype(jnp.bfloat16)
    out_ref[...] = (
        jnp.dot(x, w_ref[...], preferred_element_type=jnp.float32) + b_ref[...]
    )


@functools.lru_cache(maxsize=None)
def _build_mm_img():
    return pl.pallas_call(
        _mm_img_body,
        grid=(IMG_BLOCKS,),
        in_specs=[
            pl.BlockSpec((BLK, D_MODEL), lambda j: (j, 0)),
            pl.BlockSpec((D_MODEL, D_MODEL), lambda j: (0, 0)),
            pl.BlockSpec((1, D_MODEL), lambda j: (0, 0)),
        ],
        out_specs=[
            pl.BlockSpec((BLK, D_MODEL), lambda j: (j * BPB, 0)),
            pl.BlockSpec((D_MODEL, D_MODEL), lambda j: (0, 0)),
        ],
        out_shape=[
            jax.ShapeDtypeStruct((OUT_ROWS, D_MODEL), jnp.float32),
            jax.ShapeDtypeStruct((D_MODEL, D_MODEL), jnp.bfloat16),
        ],
        compiler_params=pltpu.CompilerParams(
            dimension_semantics=("arbitrary",),
        ),
    )


@functools.lru_cache(maxsize=None)
def _build_mm_emb(part: int):
    def out_map(j, part=part):
        return ((part * 2 + j // 8) * BPB + 1 + j % 8, 0)

    return pl.pallas_call(
        _mm_emb_body,
        grid=(EMB_BLOCKS_P,),
        in_specs=[
            pl.BlockSpec(memory_space=pl.ANY),
            pl.BlockSpec((BLK, D_MODEL), lambda j: (j, 0)),
            pl.BlockSpec((D_MODEL, D_MODEL), lambda j: (0, 0)),
            pl.BlockSpec((1, D_MODEL), lambda j: (0, 0)),
        ],
        out_specs=pl.BlockSpec((BLK, D_MODEL), out_map),
        out_shape=jax.ShapeDtypeStruct((OUT_ROWS, D_MODEL), jnp.float32),
        input_output_aliases={0: 0},
        compiler_params=pltpu.CompilerParams(
            dimension_semantics=("arbitrary",),
        ),
    )


def kernel(input_ids, image_features, table, W, b):
    ids_flat = input_ids.reshape(NTOK)
    embs = [_build_gather(q)(ids_flat, table) for q in range(NSPLIT)]
    img2d = image_features.reshape(BATCH * IMG_LEN, D_MODEL)
    b2d = b.reshape(1, D_MODEL)
    out, w_bf = _build_mm_img()(img2d, W, b2d)
    for q in range(NSPLIT):
        out = _build_mm_emb(q)(out, embs[q], w_bf, b2d)
    return out.reshape(BATCH, IMG_LEN + SEQ, D_MODEL)
